# baseline (speedup 1.0000x reference)
import functools

import jax
import jax.numpy as jnp
from jax import lax
from jax.experimental import pallas as pl
from jax.experimental.pallas import tpu as pltpu
from jax.experimental.pallas import tpu_sc as plsc

N = 10000
E = 320000
D = 128
BN_EPS = 1e-5

NC = 2
NS = 16
NW = NC * NS
EDGES_PER_TILE = E // NW
CHUNK = 80
NCHUNK = EDGES_PER_TILE // CHUNK
GROUP = 5
NGROUP = NCHUNK // GROUP
NBUF = 3
ACC_ROWS = N
ROW_STRIDE = 624
ROW_WIN = 640
ZROWS = 8
DEI_BLOCKS = 79
DEI_BATCH = 1024
DEI_NBATCH = 10


def _sc_segment_sum(features, edge_index):
    mesh = plsc.VectorSubcoreMesh(core_axis_name="c", subcore_axis_name="s")

    @functools.partial(
        pl.kernel,
        out_type=(jax.ShapeDtypeStruct((NC, N, D), jnp.float32),
                  jax.ShapeDtypeStruct((E,), jnp.int32),
                  jax.ShapeDtypeStruct((E,), jnp.int32)),
        mesh=mesh,
        scratch_types=[
            pltpu.VMEM((2, GROUP, CHUNK), jnp.int32),
            pltpu.VMEM((2, GROUP, CHUNK), jnp.int32),
            pltpu.VMEM((2, 2, DEI_BATCH), jnp.int32),
            pltpu.VMEM((NBUF, CHUNK, D), jnp.float32),
            pltpu.VMEM((ZROWS, D), jnp.float32),
            pltpu.VMEM_SHARED((ACC_ROWS, D), jnp.float32),
            pltpu.SemaphoreType.DMA((NBUF,)),
            pltpu.SemaphoreType.DMA((NBUF,)),
            pltpu.SemaphoreType.DMA((2,)),
            pltpu.SemaphoreType.DMA((2,)),
            pltpu.SemaphoreType.DMA((2,)),
            pltpu.SemaphoreType.DMA,
        ],
    )
    def k(features_hbm, ei_hbm, out_hbm, src_hbm, dst_hbm,
          src_idx, dst_idx, dei, rows, zbuf, acc,
          gsem, ssem, isem, din, dout, zsem):
        cid = lax.axis_index("c")
        sid = lax.axis_index("s")
        wid = cid * NS + sid
        ebase = wid * EDGES_PER_TILE
        al = pl.multiple_of((ebase // 128) * 128, 128)

        def _dei_sizes(kk):
            return DEI_BATCH if kk < DEI_NBATCH - 1 else (
                DEI_BLOCKS * 128 - (DEI_NBATCH - 1) * DEI_BATCH)

        def _in_desc(kk):
            sz = _dei_sizes(kk)
            off = pl.multiple_of(al + kk * DEI_BATCH, 128)
            return (ei_hbm.at[:, pl.ds(off, sz)],
                    dei.at[kk % 2, :, pl.ds(0, sz)], din.at[kk % 2])

        def _out_descs(kk):
            sz = _dei_sizes(kk)
            off = pl.multiple_of(al + kk * DEI_BATCH, 128)
            return ((dei.at[kk % 2, 0, pl.ds(0, sz)],
                     src_hbm.at[pl.ds(off, sz)], dout.at[kk % 2]),
                    (dei.at[kk % 2, 1, pl.ds(0, sz)],
                     dst_hbm.at[pl.ds(off, sz)], dout.at[kk % 2]))

        pltpu.async_copy(*_in_desc(0))
        pltpu.async_copy(*_in_desc(1))
        for kk in range(DEI_NBATCH):
            pltpu.make_async_copy(*_in_desc(kk)).wait()
            d0, d1 = _out_descs(kk)
            pltpu.async_copy(*d0)
            pltpu.async_copy(*d1)
            if kk + 2 < DEI_NBATCH:
                pltpu.make_async_copy(*d0).wait()
                pltpu.make_async_copy(*d1).wait()
                pltpu.async_copy(*_in_desc(kk + 2))

        def zstore(i, carry):
            zbuf[i // (D // 16), pl.ds((i % (D // 16)) * 16, 16)] = (
                jnp.zeros((16,), jnp.float32))
            return carry

        lax.fori_loop(0, ZROWS * (D // 16), zstore, 0)
        base = sid * ROW_STRIDE
        for q in range(ROW_WIN // ZROWS):
            pltpu.async_copy(zbuf, acc.at[pl.ds(base + q * ZROWS, ZROWS)],
                             zsem)
        for q in range(ROW_WIN // ZROWS):
            pltpu.make_async_copy(zbuf, acc.at[pl.ds(base + q * ZROWS, ZROWS)],
                                  zsem).wait()

        for kk in range(DEI_NBATCH - 2, DEI_NBATCH):
            d0, d1 = _out_descs(kk)
            pltpu.make_async_copy(*d0).wait()
            pltpu.make_async_copy(*d1).wait()

        def _stage_group(g, sem):
            for rr in range(GROUP):
                off = pl.multiple_of(ebase + g * (GROUP * CHUNK) + rr * CHUNK,
                                     16)
                pltpu.async_copy(src_hbm.at[pl.ds(off, CHUNK)],
                                 src_idx.at[g % 2, rr], sem)
                pltpu.async_copy(dst_hbm.at[pl.ds(off, CHUNK)],
                                 dst_idx.at[g % 2, rr], sem)

        def _wait_group(g, sem):
            for rr in range(GROUP):
                off = pl.multiple_of(ebase + g * (GROUP * CHUNK) + rr * CHUNK,
                                     16)
                pltpu.make_async_copy(src_hbm.at[pl.ds(off, CHUNK)],
                                      src_idx.at[g % 2, rr], sem).wait()
                pltpu.make_async_copy(dst_hbm.at[pl.ds(off, CHUNK)],
                                      dst_idx.at[g % 2, rr], sem).wait()

        _stage_group(0, isem.at[0])
        _wait_group(0, isem.at[0])
        plsc.subcore_barrier()

        for w in range(NBUF - 1):
            pltpu.async_copy(features_hbm.at[src_idx.at[0, w]], rows.at[w],
                             gsem.at[w])

        def body(j, carry):
            g = j // GROUP
            r = j % GROUP
            pg = g % 2
            pj = j % NBUF
            nxt = j + NBUF - 1

            @pl.when(jnp.logical_and(r == 0, g < NGROUP - 1))
            def _stage_next_group():
                _stage_group(g + 1, isem.at[(g + 1) % 2])

            @pl.when(jnp.logical_and(r == GROUP - NBUF + 1, g < NGROUP - 1))
            def _wait_next_group():
                _wait_group(g + 1, isem.at[(g + 1) % 2])

            @pl.when(nxt < NCHUNK)
            def _fire_next_gather():
                @pl.when(j >= 1)
                def _reuse_wait():
                    pltpu.make_async_copy(rows.at[nxt % NBUF],
                                          acc.at[dst_idx.at[pg, r]],
                                          ssem.at[nxt % NBUF]).wait()
                pltpu.async_copy(
                    features_hbm.at[src_idx.at[(nxt // GROUP) % 2,
                                               nxt % GROUP]],
                    rows.at[nxt % NBUF], gsem.at[nxt % NBUF])

            pltpu.make_async_copy(features_hbm.at[src_idx.at[pg, r]],
                                  rows.at[pj], gsem.at[pj]).wait()
            pltpu.async_copy(rows.at[pj], acc.at[dst_idx.at[pg, r]],
                             ssem.at[pj], add=True)
            return carry

        lax.fori_loop(0, NCHUNK, body, 0)
        for c in range(NCHUNK - NBUF, NCHUNK):
            pltpu.make_async_copy(rows.at[c % NBUF],
                                  acc.at[dst_idx.at[(c // GROUP) % 2,
                                                    c % GROUP]],
                                  ssem.at[c % NBUF]).wait()

        plsc.subcore_barrier()
        pltpu.sync_copy(acc.at[pl.ds(base, ROW_WIN)],
                        out_hbm.at[cid, pl.ds(base, ROW_WIN)])

    return k(features, edge_index)


BLK = 2000
NB = N // BLK


def _tc_body(f_ref, p_ref, w_ref, b_ref, g_ref, be_ref, o_ref,
             y_scr, s_ref, ss_ref):
    ph = pl.program_id(0)
    i = pl.program_id(1)

    @pl.when(ph == 0)
    def _phase0():
        f = f_ref[...]
        h = f + (p_ref[0] + p_ref[1])
        y = (jnp.dot(h, w_ref[...], preferred_element_type=jnp.float32)
             + b_ref[...])
        y_scr[pl.ds(i * BLK, BLK), :] = y

        @pl.when(i == 0)
        def _init():
            s_ref[...] = jnp.zeros_like(s_ref)
            ss_ref[...] = jnp.zeros_like(ss_ref)

        s_ref[...] += jnp.sum(y, axis=0, keepdims=True)
        ss_ref[...] += jnp.sum(y * y, axis=0, keepdims=True)
        o_ref[...] = f

    @pl.when(ph == 1)
    def _phase1():
        mean = s_ref[...] * (1.0 / N)
        var = ss_ref[...] * (1.0 / N) - mean * mean
        y = y_scr[pl.ds(i * BLK, BLK), :]
        yn = (y - mean) * lax.rsqrt(var + BN_EPS) * g_ref[...] + be_ref[...]
        o_ref[...] = f_ref[...] + jnp.maximum(yn, 0.0)


def kernel(features, edge_index, norm, W, b, gamma, beta):
    del norm
    partials, _, _ = _sc_segment_sum(features, edge_index)
    return pl.pallas_call(
        _tc_body,
        grid=(2, NB),
        in_specs=[
            pl.BlockSpec((BLK, D), lambda p, i: (i, 0)),
            pl.BlockSpec((NC, BLK, D), lambda p, i: (0, i * (1 - p), 0)),
            pl.BlockSpec((D, D), lambda p, i: (0, 0)),
            pl.BlockSpec((1, D), lambda p, i: (0, 0)),
            pl.BlockSpec((1, D), lambda p, i: (0, 0)),
            pl.BlockSpec((1, D), lambda p, i: (0, 0)),
        ],
        out_specs=pl.BlockSpec((BLK, D), lambda p, i: (i * p, 0)),
        scratch_shapes=[
            pltpu.VMEM((N, D), jnp.float32),
            pltpu.VMEM((1, D), jnp.float32),
            pltpu.VMEM((1, D), jnp.float32),
        ],
        out_shape=jax.ShapeDtypeStruct((N, D), jnp.float32),
    )(features, partials, W, b.reshape(1, D), gamma.reshape(1, D),
      beta.reshape(1, D))


# speedup vs baseline: 1.0112x; 1.0112x over previous
import functools

import jax
import jax.numpy as jnp
from jax import lax
from jax.experimental import pallas as pl
from jax.experimental.pallas import tpu as pltpu
from jax.experimental.pallas import tpu_sc as plsc

N = 10000
E = 320000
D = 128
BN_EPS = 1e-5

NC = 2
NS = 16
NW = NC * NS
EDGES_PER_TILE = E // NW
CHUNK = 80
NCHUNK = EDGES_PER_TILE // CHUNK
GROUP = 5
NGROUP = NCHUNK // GROUP
NBUF = 3
ACC_ROWS = N
ROW_STRIDE = 624
ROW_WIN = 640
ZROWS = 8
DEI_BLOCKS = 79
DEI_BATCH = 1024
DEI_NBATCH = 10


def _sc_segment_sum(features, edge_index):
    mesh = plsc.VectorSubcoreMesh(core_axis_name="c", subcore_axis_name="s")

    @functools.partial(
        pl.kernel,
        out_type=(jax.ShapeDtypeStruct((NC, N, D), jnp.float32),
                  jax.ShapeDtypeStruct((E,), jnp.int32),
                  jax.ShapeDtypeStruct((E,), jnp.int32)),
        mesh=mesh,
        scratch_types=[
            pltpu.VMEM((2, GROUP, CHUNK), jnp.int32),
            pltpu.VMEM((2, GROUP, CHUNK), jnp.int32),
            pltpu.VMEM((2, 2, DEI_BATCH), jnp.int32),
            pltpu.VMEM((NBUF, CHUNK, D), jnp.float32),
            pltpu.VMEM((ZROWS, D), jnp.float32),
            pltpu.VMEM_SHARED((ACC_ROWS, D), jnp.float32),
            pltpu.SemaphoreType.DMA((NBUF,)),
            pltpu.SemaphoreType.DMA((NBUF,)),
            pltpu.SemaphoreType.DMA((2,)),
            pltpu.SemaphoreType.DMA((2,)),
            pltpu.SemaphoreType.DMA((2,)),
            pltpu.SemaphoreType.DMA,
        ],
    )
    def k(features_hbm, ei_hbm, out_hbm, src_hbm, dst_hbm,
          src_idx, dst_idx, dei, rows, zbuf, acc,
          gsem, ssem, isem, din, dout, zsem):
        cid = lax.axis_index("c")
        sid = lax.axis_index("s")
        wid = cid * NS + sid
        ebase = wid * EDGES_PER_TILE
        al = pl.multiple_of((ebase // 128) * 128, 128)

        def _dei_sizes(kk):
            return DEI_BATCH if kk < DEI_NBATCH - 1 else (
                DEI_BLOCKS * 128 - (DEI_NBATCH - 1) * DEI_BATCH)

        def _in_desc(kk):
            sz = _dei_sizes(kk)
            off = pl.multiple_of(al + kk * DEI_BATCH, 128)
            return (ei_hbm.at[:, pl.ds(off, sz)],
                    dei.at[kk % 2, :, pl.ds(0, sz)], din.at[kk % 2])

        def _out_descs(kk):
            sz = _dei_sizes(kk)
            off = pl.multiple_of(al + kk * DEI_BATCH, 128)
            return ((dei.at[kk % 2, 0, pl.ds(0, sz)],
                     src_hbm.at[pl.ds(off, sz)], dout.at[kk % 2]),
                    (dei.at[kk % 2, 1, pl.ds(0, sz)],
                     dst_hbm.at[pl.ds(off, sz)], dout.at[kk % 2]))

        pltpu.async_copy(*_in_desc(0))
        pltpu.async_copy(*_in_desc(1))
        for kk in range(DEI_NBATCH):
            pltpu.make_async_copy(*_in_desc(kk)).wait()
            d0, d1 = _out_descs(kk)
            pltpu.async_copy(*d0)
            pltpu.async_copy(*d1)
            if kk + 2 < DEI_NBATCH:
                pltpu.make_async_copy(*d0).wait()
                pltpu.make_async_copy(*d1).wait()
                pltpu.async_copy(*_in_desc(kk + 2))

        def zstore(i, carry):
            zbuf[i // (D // 16), pl.ds((i % (D // 16)) * 16, 16)] = (
                jnp.zeros((16,), jnp.float32))
            return carry

        lax.fori_loop(0, ZROWS * (D // 16), zstore, 0)
        base = sid * ROW_STRIDE
        for q in range(ROW_WIN // ZROWS):
            pltpu.async_copy(zbuf, acc.at[pl.ds(base + q * ZROWS, ZROWS)],
                             zsem)
        for q in range(ROW_WIN // ZROWS):
            pltpu.make_async_copy(zbuf, acc.at[pl.ds(base + q * ZROWS, ZROWS)],
                                  zsem).wait()

        for kk in range(DEI_NBATCH - 2, DEI_NBATCH):
            d0, d1 = _out_descs(kk)
            pltpu.make_async_copy(*d0).wait()
            pltpu.make_async_copy(*d1).wait()

        def _stage_group(g, sem):
            for rr in range(GROUP):
                off = pl.multiple_of(ebase + g * (GROUP * CHUNK) + rr * CHUNK,
                                     16)
                pltpu.async_copy(src_hbm.at[pl.ds(off, CHUNK)],
                                 src_idx.at[g % 2, rr], sem)
                pltpu.async_copy(dst_hbm.at[pl.ds(off, CHUNK)],
                                 dst_idx.at[g % 2, rr], sem)

        def _wait_group(g, sem):
            for rr in range(GROUP):
                off = pl.multiple_of(ebase + g * (GROUP * CHUNK) + rr * CHUNK,
                                     16)
                pltpu.make_async_copy(src_hbm.at[pl.ds(off, CHUNK)],
                                      src_idx.at[g % 2, rr], sem).wait()
                pltpu.make_async_copy(dst_hbm.at[pl.ds(off, CHUNK)],
                                      dst_idx.at[g % 2, rr], sem).wait()

        _stage_group(0, isem.at[0])
        _wait_group(0, isem.at[0])
        plsc.subcore_barrier()

        for w in range(NBUF - 1):
            pltpu.async_copy(features_hbm.at[src_idx.at[0, w]], rows.at[w],
                             gsem.at[w])

        def body(j, carry):
            g = j // GROUP
            r = j % GROUP
            pg = g % 2
            pj = j % NBUF
            nxt = j + NBUF - 1

            @pl.when(jnp.logical_and(r == 0, g < NGROUP - 1))
            def _stage_next_group():
                _stage_group(g + 1, isem.at[(g + 1) % 2])

            @pl.when(jnp.logical_and(r == GROUP - NBUF + 1, g < NGROUP - 1))
            def _wait_next_group():
                _wait_group(g + 1, isem.at[(g + 1) % 2])

            @pl.when(nxt < NCHUNK)
            def _fire_next_gather():
                @pl.when(j >= 1)
                def _reuse_wait():
                    pltpu.make_async_copy(rows.at[nxt % NBUF],
                                          acc.at[dst_idx.at[pg, r]],
                                          ssem.at[nxt % NBUF]).wait()
                pltpu.async_copy(
                    features_hbm.at[src_idx.at[(nxt // GROUP) % 2,
                                               nxt % GROUP]],
                    rows.at[nxt % NBUF], gsem.at[nxt % NBUF])

            pltpu.make_async_copy(features_hbm.at[src_idx.at[pg, r]],
                                  rows.at[pj], gsem.at[pj]).wait()
            pltpu.async_copy(rows.at[pj], acc.at[dst_idx.at[pg, r]],
                             ssem.at[pj], add=True)
            return carry

        lax.fori_loop(0, NCHUNK, body, 0)
        for c in range(NCHUNK - NBUF, NCHUNK):
            pltpu.make_async_copy(rows.at[c % NBUF],
                                  acc.at[dst_idx.at[(c // GROUP) % 2,
                                                    c % GROUP]],
                                  ssem.at[c % NBUF]).wait()

        plsc.subcore_barrier()
        pltpu.sync_copy(acc.at[pl.ds(base, ROW_WIN)],
                        out_hbm.at[cid, pl.ds(base, ROW_WIN)])

    return k(features, edge_index)


def _tc_body(f_ref, p_ref, w_ref, b_ref, g_ref, be_ref, o_ref):
    f = f_ref[...]
    h = f + (p_ref[0] + p_ref[1])
    y = jnp.dot(h, w_ref[...], preferred_element_type=jnp.float32) + b_ref[...]
    mean = jnp.mean(y, axis=0, keepdims=True)
    c = y - mean
    var = jnp.mean(c * c, axis=0, keepdims=True)
    yn = c * lax.rsqrt(var + BN_EPS) * g_ref[...] + be_ref[...]
    o_ref[...] = f + jnp.maximum(yn, 0.0)


def kernel(features, edge_index, norm, W, b, gamma, beta):
    del norm
    partials, _, _ = _sc_segment_sum(features, edge_index)
    return pl.pallas_call(
        _tc_body,
        out_shape=jax.ShapeDtypeStruct((N, D), jnp.float32),
    )(features, partials, W, b.reshape(1, D), gamma.reshape(1, D),
      beta.reshape(1, D))
